# baseline (speedup 1.0000x reference)
import functools

import jax
import jax.numpy as jnp
from jax import lax
from jax.experimental import pallas as pl
from jax.experimental.pallas import tpu as pltpu
from jax.experimental.pallas import tpu_sc as plsc

N = 10000
E = 320000
NPAD = 10240
EPAD = 327680
F_IN, F_HID, F_OUT = 128, 256, 64
NC, NS = 2, 16
CHUNK = 64
NBUF = 5

_mesh = plsc.VectorSubcoreMesh(core_axis_name="c", subcore_axis_name="s")


def _zero_vec_ref(ref, rows, cols):
    def body_r(r, _):
        def body_c(k, _):
            ref[r, pl.ds(k * 16, 16)] = jnp.zeros((16,), jnp.float32)
            return 0
        return lax.fori_loop(0, cols // 16, body_c, 0)
    lax.fori_loop(0, rows, body_r, 0)


@functools.partial(
    pl.kernel,
    out_type=jax.ShapeDtypeStruct((NC, NPAD), jnp.float32),
    mesh=_mesh,
    scratch_types=[
        pltpu.VMEM((EPAD // (NC * NS) // CHUNK, CHUNK), jnp.int32),
        pltpu.VMEM((CHUNK,), jnp.float32),
        pltpu.VMEM((NPAD // NS,), jnp.float32),
        pltpu.VMEM_SHARED((NPAD,), jnp.float32),
    ] + [pltpu.SemaphoreType.DMA] * NBUF,
)
def _deg_kernel(dst2_hbm, out_hbm, didx_v, ones_v, zrow_v, hist_sh, *ssems):
    c = lax.axis_index("c")
    s = lax.axis_index("s")
    zn = NPAD // NS
    nch = EPAD // (NC * NS) // CHUNK
    row0 = pl.multiple_of((c * NS + s) * nch, 8)

    def zb(k, _):
        zrow_v[pl.ds(k * 16, 16)] = jnp.zeros((16,), jnp.float32)
        return 0
    lax.fori_loop(0, zn // 16, zb, 0)

    def ob(k, _):
        ones_v[pl.ds(k * 16, 16)] = jnp.ones((16,), jnp.float32)
        return 0
    lax.fori_loop(0, CHUNK // 16, ob, 0)

    pltpu.sync_copy(dst2_hbm.at[pl.ds(row0, nch)], didx_v)
    pltpu.sync_copy(zrow_v, hist_sh.at[pl.ds(s * zn, zn)])
    plsc.subcore_barrier()

    def sc_start(j, b):
        pltpu.async_copy(ones_v, hist_sh.at[didx_v.at[j]], ssems[b], add=True)

    def sc_wait(b):
        pltpu.make_async_copy(ones_v, hist_sh.at[didx_v.at[0]], ssems[b]).wait()

    for b in range(NBUF):
        sc_start(b, b)

    def step(g, _):
        for b in range(NBUF):
            j = NBUF * g + b
            sc_wait(b)
            sc_start(j + NBUF, b)
        return 0
    lax.fori_loop(0, nch // NBUF - 1, step, 0)
    for b in range(NBUF):
        sc_wait(b)

    plsc.subcore_barrier()
    pltpu.sync_copy(hist_sh.at[pl.ds(s * zn, zn)], out_hbm.at[c, pl.ds(s * zn, zn)])


def _edge_body(src_hbm, dst2_hbm, h_hbm, out_view, acc_sh,
               sidxg, didxg, rowbufs, gsems, ssems, isems,
               base, nch, cpg, s, fh):
    groups = nch // cpg
    gb = cpg * CHUNK

    def i_start(g, p):
        e0 = pl.multiple_of(base + g * gb, 8)
        pltpu.async_copy(src_hbm.at[pl.ds(e0, gb)], sidxg[p], isems[p])
        r0 = pl.multiple_of((base + g * gb) // CHUNK, 8)
        pltpu.async_copy(dst2_hbm.at[pl.ds(r0, cpg)], didxg[p], isems[p])

    def i_wait(p):
        pltpu.make_async_copy(src_hbm.at[pl.ds(0, gb)], sidxg[p], isems[p]).wait()
        pltpu.make_async_copy(dst2_hbm.at[pl.ds(0, cpg)], didxg[p], isems[p]).wait()

    i_start(0, 0)
    i_start(1, 1)

    _zero_vec_ref(rowbufs[0], CHUNK, fh)
    rz = NPAD // NS
    for k in range(rz // CHUNK):
        pltpu.sync_copy(rowbufs[0],
                        acc_sh.at[pl.ds(pl.multiple_of(s * rz + k * CHUNK, 8), CHUNK)])
    plsc.subcore_barrier()

    def process_group(p):
        sg, dg = sidxg[p], didxg[p]

        def g_start(t, b):
            pltpu.async_copy(h_hbm.at[sg.at[pl.ds(t * CHUNK, CHUNK)]],
                             rowbufs[b], gsems[b])

        def g_wait(b):
            pltpu.make_async_copy(h_hbm.at[sg.at[pl.ds(0, CHUNK)]],
                                  rowbufs[b], gsems[b]).wait()

        def s_wait(b):
            pltpu.make_async_copy(rowbufs[b], acc_sh.at[dg.at[0]], ssems[b]).wait()

        for b in range(NBUF):
            g_start(b, b)
        for t in range(cpg):
            b = t % NBUF
            g_wait(b)
            pltpu.async_copy(rowbufs[b], acc_sh.at[dg.at[t]], ssems[b], add=True)
            if t + NBUF < cpg:
                s_wait(b)
                g_start(t + NBUF, b)
        for t in range(max(cpg - NBUF, 0), cpg):
            s_wait(t % NBUF)

    def step2(gg, _):
        for p in range(2):
            g = 2 * gg + p
            i_wait(p)
            process_group(p)

            @pl.when(gg < groups // 2 - 1)
            def _():
                i_start(g + 2, p)
        return 0
    lax.fori_loop(0, groups // 2, step2, 0)

    plsc.subcore_barrier()
    r0 = pl.multiple_of(s * rz, 8)
    pltpu.sync_copy(acc_sh.at[pl.ds(r0, rz)], out_view.at[pl.ds(r0, rz)])


def _edge_scratch(fh, cpg):
    return ([pltpu.VMEM((cpg * CHUNK,), jnp.int32)] * 2
            + [pltpu.VMEM((cpg, CHUNK), jnp.int32)] * 2
            + [pltpu.VMEM((CHUNK, fh), jnp.float32)] * NBUF
            + [pltpu.SemaphoreType.DMA] * (2 * NBUF + 2)
            + [pltpu.VMEM_SHARED((NPAD, fh), jnp.float32)])


def _split_scratch(rest):
    sidxg = rest[0:2]
    didxg = rest[2:4]
    rowbufs = rest[4:4 + NBUF]
    r = 4 + NBUF
    gsems = rest[r:r + NBUF]
    ssems = rest[r + NBUF:r + 2 * NBUF]
    isems = rest[r + 2 * NBUF:r + 2 * NBUF + 2]
    acc_sh = rest[-1]
    return sidxg, didxg, rowbufs, gsems, ssems, isems, acc_sh


def _make_scatter1():
    fh = F_HID // 2
    nch = EPAD // NS // CHUNK
    cpg = 16

    @functools.partial(
        pl.kernel,
        out_type=[jax.ShapeDtypeStruct((NPAD, fh), jnp.float32),
                  jax.ShapeDtypeStruct((NPAD, fh), jnp.float32)],
        mesh=_mesh,
        scratch_types=_edge_scratch(fh, cpg),
    )
    def k(src_hbm, dst2_hbm, h0_hbm, h1_hbm, s0_hbm, s1_hbm, *rest):
        sidxg, didxg, rowbufs, gsems, ssems, isems, acc_sh = _split_scratch(rest)
        c = lax.axis_index("c")
        s = lax.axis_index("s")
        base = s * (EPAD // NS)

        @pl.when(c == 0)
        def _():
            _edge_body(src_hbm, dst2_hbm, h0_hbm, s0_hbm, acc_sh,
                       sidxg, didxg, rowbufs, gsems, ssems, isems,
                       base, nch, cpg, s, fh)

        @pl.when(c == 1)
        def _():
            _edge_body(src_hbm, dst2_hbm, h1_hbm, s1_hbm, acc_sh,
                       sidxg, didxg, rowbufs, gsems, ssems, isems,
                       base, nch, cpg, s, fh)

    return k


def _make_scatter2():
    fh = F_OUT
    nch = EPAD // (NC * NS) // CHUNK
    cpg = 8

    @functools.partial(
        pl.kernel,
        out_type=jax.ShapeDtypeStruct((NC, NPAD, fh), jnp.float32),
        mesh=_mesh,
        scratch_types=_edge_scratch(fh, cpg),
        compiler_params=pltpu.CompilerParams(use_tc_tiling_on_sc=False),
    )
    def k(src_hbm, dst2_hbm, h_hbm, s2_hbm, *rest):
        sidxg, didxg, rowbufs, gsems, ssems, isems, acc_sh = _split_scratch(rest)
        c = lax.axis_index("c")
        s = lax.axis_index("s")
        base = (c * NS + s) * (EPAD // (NC * NS))
        _edge_body(src_hbm, dst2_hbm, h_hbm, s2_hbm.at[c], acc_sh,
                   sidxg, didxg, rowbufs, gsems, ssems, isems,
                   base, nch, cpg, s, fh)

    return k


_scatter1 = _make_scatter1()
_scatter2 = _make_scatter2()


RB = 1000


def _tc1_body(x_ref, w1_ref, degp_ref, h0_ref, h1_ref):
    dinv = lax.rsqrt(1.0 + degp_ref[0] + degp_ref[1])
    hs = jnp.dot(x_ref[...], w1_ref[...], preferred_element_type=jnp.float32)
    hs = hs * dinv
    h0_ref[...] = hs[:, :F_HID // 2]
    h1_ref[...] = hs[:, F_HID // 2:]


def _tc1(x, w1, degp):
    half = F_HID // 2
    return pl.pallas_call(
        _tc1_body,
        grid=(N // RB,),
        in_specs=[
            pl.BlockSpec((RB, F_IN), lambda i: (i, 0)),
            pl.BlockSpec((F_IN, F_HID), lambda i: (0, 0)),
            pl.BlockSpec((NC, RB, 1), lambda i: (0, i, 0)),
        ],
        out_specs=[
            pl.BlockSpec((RB, half), lambda i: (i, 0)),
            pl.BlockSpec((RB, half), lambda i: (i, 0)),
        ],
        out_shape=[jax.ShapeDtypeStruct((NPAD, half), jnp.float32),
                   jax.ShapeDtypeStruct((NPAD, half), jnp.float32)],
    )(x, w1, degp)


def _tc2_body(s0_ref, s1_ref, h0_ref, h1_ref, degp_ref, b1_ref, w2_ref, out_ref):
    dinv = lax.rsqrt(1.0 + degp_ref[0] + degp_ref[1])
    half = F_HID // 2
    a0 = jax.nn.relu(dinv * (s0_ref[...] + h0_ref[...]) + b1_ref[0, :half])
    a1 = jax.nn.relu(dinv * (s1_ref[...] + h1_ref[...]) + b1_ref[0, half:])
    hs2 = (jnp.dot(a0, w2_ref[:half, :], preferred_element_type=jnp.float32)
           + jnp.dot(a1, w2_ref[half:, :], preferred_element_type=jnp.float32))
    out_ref[...] = hs2 * dinv


def _tc2(s0, s1, h0, h1, degp, b1, w2):
    half = F_HID // 2
    return pl.pallas_call(
        _tc2_body,
        grid=(N // RB,),
        in_specs=[
            pl.BlockSpec((RB, half), lambda i: (i, 0)),
            pl.BlockSpec((RB, half), lambda i: (i, 0)),
            pl.BlockSpec((RB, half), lambda i: (i, 0)),
            pl.BlockSpec((RB, half), lambda i: (i, 0)),
            pl.BlockSpec((NC, RB, 1), lambda i: (0, i, 0)),
            pl.BlockSpec((1, F_HID), lambda i: (0, 0)),
            pl.BlockSpec((F_HID, F_OUT), lambda i: (0, 0)),
        ],
        out_specs=pl.BlockSpec((RB, F_OUT), lambda i: (i, 0)),
        out_shape=jax.ShapeDtypeStruct((NPAD, F_OUT), jnp.float32),
    )(s0, s1, h0, h1, degp, b1, w2)


def _tc3_body(s2_ref, h2_ref, degp_ref, b2_ref, out_ref):
    dinv = lax.rsqrt(1.0 + degp_ref[0] + degp_ref[1])
    out_ref[...] = dinv * (s2_ref[0] + s2_ref[1] + h2_ref[...]) + b2_ref[0, :]


def _tc3(s2, h2, degp, b2):
    return pl.pallas_call(
        _tc3_body,
        grid=(N // RB,),
        in_specs=[
            pl.BlockSpec((NC, RB, F_OUT), lambda i: (0, i, 0)),
            pl.BlockSpec((RB, F_OUT), lambda i: (i, 0)),
            pl.BlockSpec((NC, RB, 1), lambda i: (0, i, 0)),
            pl.BlockSpec((1, F_OUT), lambda i: (0, 0)),
        ],
        out_specs=pl.BlockSpec((RB, F_OUT), lambda i: (i, 0)),
        out_shape=jax.ShapeDtypeStruct((N, F_OUT), jnp.float32),
    )(s2, h2, degp, b2)


@jax.jit
def kernel(x, edge_index, W1, b1, W2, b2):
    src = edge_index[0].astype(jnp.int32)
    dst = edge_index[1].astype(jnp.int32)

    pad = N + jnp.arange(EPAD - E, dtype=jnp.int32) % (NPAD - N)
    src_p = jnp.concatenate([src, pad])
    dst_p = jnp.concatenate([dst, pad])
    dst2 = dst_p.reshape(EPAD // CHUNK, CHUNK)

    degp = _deg_kernel(dst2).reshape(NC, NPAD, 1)
    h0, h1 = _tc1(x, W1, degp)
    s0, s1 = _scatter1(src_p, dst2, h0, h1)
    hs2 = _tc2(s0, s1, h0, h1, degp, b1.reshape(1, F_HID), W2)
    s2 = _scatter2(src_p, dst2, hs2)
    return _tc3(s2, hs2, degp, b2.reshape(1, F_OUT))


# speedup vs baseline: 1.1415x; 1.1415x over previous
import functools

import jax
import jax.numpy as jnp
from jax import lax
from jax.experimental import pallas as pl
from jax.experimental.pallas import tpu as pltpu
from jax.experimental.pallas import tpu_sc as plsc

N = 10000
E = 320000
NPAD = 10240
EPAD = 327680
F_IN, F_HID, F_OUT = 128, 256, 64
NC, NS = 2, 16
CHUNK = 64
NBUF = 4

_mesh = plsc.VectorSubcoreMesh(core_axis_name="c", subcore_axis_name="s")


def _zero_vec_ref(ref, rows, cols):
    def body_r(r, _):
        def body_c(k, _):
            ref[r, pl.ds(k * 16, 16)] = jnp.zeros((16,), jnp.float32)
            return 0
        return lax.fori_loop(0, cols // 16, body_c, 0)
    lax.fori_loop(0, rows, body_r, 0)


@functools.partial(
    pl.kernel,
    out_type=jax.ShapeDtypeStruct((NC, NPAD), jnp.float32),
    mesh=_mesh,
    scratch_types=[
        pltpu.VMEM((EPAD // (NC * NS) // CHUNK, CHUNK), jnp.int32),
        pltpu.VMEM((CHUNK,), jnp.float32),
        pltpu.VMEM((NPAD // NS,), jnp.float32),
        pltpu.VMEM_SHARED((NPAD,), jnp.float32),
    ] + [pltpu.SemaphoreType.DMA] * NBUF,
)
def _deg_kernel(dst2_hbm, out_hbm, didx_v, ones_v, zrow_v, hist_sh, *ssems):
    c = lax.axis_index("c")
    s = lax.axis_index("s")
    zn = NPAD // NS
    nch = EPAD // (NC * NS) // CHUNK
    row0 = pl.multiple_of((c * NS + s) * nch, 8)

    def zb(k, _):
        zrow_v[pl.ds(k * 16, 16)] = jnp.zeros((16,), jnp.float32)
        return 0
    lax.fori_loop(0, zn // 16, zb, 0)

    def ob(k, _):
        ones_v[pl.ds(k * 16, 16)] = jnp.ones((16,), jnp.float32)
        return 0
    lax.fori_loop(0, CHUNK // 16, ob, 0)

    pltpu.sync_copy(dst2_hbm.at[pl.ds(row0, nch)], didx_v)
    pltpu.sync_copy(zrow_v, hist_sh.at[pl.ds(s * zn, zn)])
    plsc.subcore_barrier()

    def sc_start(j, b):
        pltpu.async_copy(ones_v, hist_sh.at[didx_v.at[j]], ssems[b], add=True)

    def sc_wait(b):
        pltpu.make_async_copy(ones_v, hist_sh.at[didx_v.at[0]], ssems[b]).wait()

    for b in range(NBUF):
        sc_start(b, b)

    def step(g, _):
        for b in range(NBUF):
            j = NBUF * g + b
            sc_wait(b)
            sc_start(j + NBUF, b)
        return 0
    lax.fori_loop(0, nch // NBUF - 1, step, 0)
    for b in range(NBUF):
        sc_wait(b)

    plsc.subcore_barrier()
    pltpu.sync_copy(hist_sh.at[pl.ds(s * zn, zn)], out_hbm.at[c, pl.ds(s * zn, zn)])


def _edge_body(src_hbm, dst2_hbm, h_hbm, out_view, acc_sh,
               sidxg, didxg, rowbufs, gsems, ssems, isems,
               base, nch, cpg, s, fh):
    groups = nch // cpg
    gb = cpg * CHUNK

    def i_start(g, p):
        e0 = pl.multiple_of(base + g * gb, 8)
        pltpu.async_copy(src_hbm.at[pl.ds(e0, gb)], sidxg[p], isems[p])
        r0 = pl.multiple_of((base + g * gb) // CHUNK, 8)
        pltpu.async_copy(dst2_hbm.at[pl.ds(r0, cpg)], didxg[p], isems[p])

    def i_wait(p):
        pltpu.make_async_copy(src_hbm.at[pl.ds(0, gb)], sidxg[p], isems[p]).wait()
        pltpu.make_async_copy(dst2_hbm.at[pl.ds(0, cpg)], didxg[p], isems[p]).wait()

    i_start(0, 0)
    i_start(1, 1)

    _zero_vec_ref(rowbufs[0], CHUNK, fh)
    rz = NPAD // NS
    for k in range(rz // CHUNK):
        pltpu.sync_copy(rowbufs[0],
                        acc_sh.at[pl.ds(pl.multiple_of(s * rz + k * CHUNK, 8), CHUNK)])
    plsc.subcore_barrier()

    def g_start(sg, t, b):
        pltpu.async_copy(h_hbm.at[sg.at[pl.ds(t * CHUNK, CHUNK)]],
                         rowbufs[b], gsems[b])

    def g_wait(b):
        pltpu.make_async_copy(h_hbm.at[sidxg[0].at[pl.ds(0, CHUNK)]],
                              rowbufs[b], gsems[b]).wait()

    def s_wait(b):
        pltpu.make_async_copy(rowbufs[b], acc_sh.at[didxg[0].at[0]],
                              ssems[b]).wait()

    i_wait(0)
    i_wait(1)
    for b in range(NBUF):
        g_start(sidxg[0], b, b)

    def step2(gg, _):
        for p in range(2):
            g = 2 * gg + p
            sg, sgn, dg = sidxg[p], sidxg[1 - p], didxg[p]
            for t in range(cpg):
                b = t % NBUF
                g_wait(b)
                pltpu.async_copy(rowbufs[b], acc_sh.at[dg.at[t]],
                                 ssems[b], add=True)
                if t == cpg - NBUF:
                    @pl.when((g >= 1) & (g <= groups - 2))
                    def _():
                        i_wait(1 - p)
                s_wait(b)
                if t + NBUF < cpg:
                    g_start(sg, t + NBUF, b)
                else:
                    @pl.when(g <= groups - 2)
                    def _(tn=t + NBUF - cpg, b=b, sgn=sgn):
                        g_start(sgn, tn, b)

            @pl.when(g <= groups - 3)
            def _():
                i_start(g + 2, p)
        return 0
    lax.fori_loop(0, groups // 2, step2, 0)

    plsc.subcore_barrier()
    r0 = pl.multiple_of(s * rz, 8)
    pltpu.sync_copy(acc_sh.at[pl.ds(r0, rz)], out_view.at[pl.ds(r0, rz)])


def _edge_scratch(fh, cpg):
    return ([pltpu.VMEM((cpg * CHUNK,), jnp.int32)] * 2
            + [pltpu.VMEM((cpg, CHUNK), jnp.int32)] * 2
            + [pltpu.VMEM((CHUNK, fh), jnp.float32)] * NBUF
            + [pltpu.SemaphoreType.DMA] * (2 * NBUF + 2)
            + [pltpu.VMEM_SHARED((NPAD, fh), jnp.float32)])


def _split_scratch(rest):
    sidxg = rest[0:2]
    didxg = rest[2:4]
    rowbufs = rest[4:4 + NBUF]
    r = 4 + NBUF
    gsems = rest[r:r + NBUF]
    ssems = rest[r + NBUF:r + 2 * NBUF]
    isems = rest[r + 2 * NBUF:r + 2 * NBUF + 2]
    acc_sh = rest[-1]
    return sidxg, didxg, rowbufs, gsems, ssems, isems, acc_sh


def _make_scatter1():
    fh = F_HID // 2
    nch = EPAD // NS // CHUNK
    cpg = 32

    @functools.partial(
        pl.kernel,
        out_type=[jax.ShapeDtypeStruct((NPAD, fh), jnp.float32),
                  jax.ShapeDtypeStruct((NPAD, fh), jnp.float32)],
        mesh=_mesh,
        scratch_types=_edge_scratch(fh, cpg),
    )
    def k(src_hbm, dst2_hbm, h0_hbm, h1_hbm, s0_hbm, s1_hbm, *rest):
        sidxg, didxg, rowbufs, gsems, ssems, isems, acc_sh = _split_scratch(rest)
        c = lax.axis_index("c")
        s = lax.axis_index("s")
        base = s * (EPAD // NS)

        @pl.when(c == 0)
        def _():
            _edge_body(src_hbm, dst2_hbm, h0_hbm, s0_hbm, acc_sh,
                       sidxg, didxg, rowbufs, gsems, ssems, isems,
                       base, nch, cpg, s, fh)

        @pl.when(c == 1)
        def _():
            _edge_body(src_hbm, dst2_hbm, h1_hbm, s1_hbm, acc_sh,
                       sidxg, didxg, rowbufs, gsems, ssems, isems,
                       base, nch, cpg, s, fh)

    return k


def _make_scatter2():
    fh = F_OUT
    nch = EPAD // (NC * NS) // CHUNK
    cpg = 16

    @functools.partial(
        pl.kernel,
        out_type=jax.ShapeDtypeStruct((NC, NPAD, fh), jnp.float32),
        mesh=_mesh,
        scratch_types=_edge_scratch(fh, cpg),
        compiler_params=pltpu.CompilerParams(use_tc_tiling_on_sc=False),
    )
    def k(src_hbm, dst2_hbm, h_hbm, s2_hbm, *rest):
        sidxg, didxg, rowbufs, gsems, ssems, isems, acc_sh = _split_scratch(rest)
        c = lax.axis_index("c")
        s = lax.axis_index("s")
        base = (c * NS + s) * (EPAD // (NC * NS))
        _edge_body(src_hbm, dst2_hbm, h_hbm, s2_hbm.at[c], acc_sh,
                   sidxg, didxg, rowbufs, gsems, ssems, isems,
                   base, nch, cpg, s, fh)

    return k


_scatter1 = _make_scatter1()
_scatter2 = _make_scatter2()


RB = 1000


def _tc1_body(x_ref, w1_ref, degp_ref, h0_ref, h1_ref):
    dinv = lax.rsqrt(1.0 + degp_ref[0] + degp_ref[1])
    hs = jnp.dot(x_ref[...], w1_ref[...], preferred_element_type=jnp.float32)
    hs = hs * dinv
    h0_ref[...] = hs[:, :F_HID // 2]
    h1_ref[...] = hs[:, F_HID // 2:]


def _tc1(x, w1, degp):
    half = F_HID // 2
    return pl.pallas_call(
        _tc1_body,
        grid=(N // RB,),
        in_specs=[
            pl.BlockSpec((RB, F_IN), lambda i: (i, 0)),
            pl.BlockSpec((F_IN, F_HID), lambda i: (0, 0)),
            pl.BlockSpec((NC, RB, 1), lambda i: (0, i, 0)),
        ],
        out_specs=[
            pl.BlockSpec((RB, half), lambda i: (i, 0)),
            pl.BlockSpec((RB, half), lambda i: (i, 0)),
        ],
        out_shape=[jax.ShapeDtypeStruct((NPAD, half), jnp.float32),
                   jax.ShapeDtypeStruct((NPAD, half), jnp.float32)],
    )(x, w1, degp)


def _tc2_body(s0_ref, s1_ref, h0_ref, h1_ref, degp_ref, b1_ref, w2_ref, out_ref):
    dinv = lax.rsqrt(1.0 + degp_ref[0] + degp_ref[1])
    half = F_HID // 2
    a0 = jax.nn.relu(dinv * (s0_ref[...] + h0_ref[...]) + b1_ref[0, :half])
    a1 = jax.nn.relu(dinv * (s1_ref[...] + h1_ref[...]) + b1_ref[0, half:])
    hs2 = (jnp.dot(a0, w2_ref[:half, :], preferred_element_type=jnp.float32)
           + jnp.dot(a1, w2_ref[half:, :], preferred_element_type=jnp.float32))
    out_ref[...] = hs2 * dinv


def _tc2(s0, s1, h0, h1, degp, b1, w2):
    half = F_HID // 2
    return pl.pallas_call(
        _tc2_body,
        grid=(N // RB,),
        in_specs=[
            pl.BlockSpec((RB, half), lambda i: (i, 0)),
            pl.BlockSpec((RB, half), lambda i: (i, 0)),
            pl.BlockSpec((RB, half), lambda i: (i, 0)),
            pl.BlockSpec((RB, half), lambda i: (i, 0)),
            pl.BlockSpec((NC, RB, 1), lambda i: (0, i, 0)),
            pl.BlockSpec((1, F_HID), lambda i: (0, 0)),
            pl.BlockSpec((F_HID, F_OUT), lambda i: (0, 0)),
        ],
        out_specs=pl.BlockSpec((RB, F_OUT), lambda i: (i, 0)),
        out_shape=jax.ShapeDtypeStruct((NPAD, F_OUT), jnp.float32),
    )(s0, s1, h0, h1, degp, b1, w2)


def _tc3_body(s2_ref, h2_ref, degp_ref, b2_ref, out_ref):
    dinv = lax.rsqrt(1.0 + degp_ref[0] + degp_ref[1])
    out_ref[...] = dinv * (s2_ref[0] + s2_ref[1] + h2_ref[...]) + b2_ref[0, :]


def _tc3(s2, h2, degp, b2):
    return pl.pallas_call(
        _tc3_body,
        grid=(N // RB,),
        in_specs=[
            pl.BlockSpec((NC, RB, F_OUT), lambda i: (0, i, 0)),
            pl.BlockSpec((RB, F_OUT), lambda i: (i, 0)),
            pl.BlockSpec((NC, RB, 1), lambda i: (0, i, 0)),
            pl.BlockSpec((1, F_OUT), lambda i: (0, 0)),
        ],
        out_specs=pl.BlockSpec((RB, F_OUT), lambda i: (i, 0)),
        out_shape=jax.ShapeDtypeStruct((N, F_OUT), jnp.float32),
    )(s2, h2, degp, b2)


@jax.jit
def kernel(x, edge_index, W1, b1, W2, b2):
    src = edge_index[0].astype(jnp.int32)
    dst = edge_index[1].astype(jnp.int32)

    pad = N + jnp.arange(EPAD - E, dtype=jnp.int32) % (NPAD - N)
    src_p = jnp.concatenate([src, pad])
    dst_p = jnp.concatenate([dst, pad])
    dst2 = dst_p.reshape(EPAD // CHUNK, CHUNK)

    degp = _deg_kernel(dst2).reshape(NC, NPAD, 1)
    h0, h1 = _tc1(x, W1, degp)
    s0, s1 = _scatter1(src_p, dst2, h0, h1)
    hs2 = _tc2(s0, s1, h0, h1, degp, b1.reshape(1, F_HID), W2)
    s2 = _scatter2(src_p, dst2, hs2)
    return _tc3(s2, hs2, degp, b2.reshape(1, F_OUT))
